# baseline (speedup 1.0000x reference)
import functools

import jax
import jax.numpy as jnp
from jax import lax
from jax.experimental import pallas as pl
from jax.experimental.pallas import tpu as pltpu
from jax.experimental.pallas import tpu_sc as plsc

_NC = 2
_NS = 16
_NW = _NC * _NS
_L = 16
_SEG = 128


@functools.partial(jax.jit, static_argnums=(3, 4, 5))
def _embed_lookup(idx_flat, tok_grouped, pos_table, B, T, D):
    n_tok = B * T
    b_per_w = n_tok // _NW
    n_seg = b_per_w // _SEG
    mesh = plsc.VectorSubcoreMesh(core_axis_name="c", subcore_axis_name="s")

    @functools.partial(
        pl.kernel,
        out_type=jax.ShapeDtypeStruct((n_tok, D), jnp.float32),
        mesh=mesh,
        compiler_params=pltpu.CompilerParams(use_tc_tiling_on_sc=False),
        scratch_types=[
            pltpu.VMEM((b_per_w,), jnp.int32),
            pltpu.VMEM((b_per_w,), jnp.int32),
            pltpu.VMEM((_SEG, 8, D), jnp.float32),
            pltpu.VMEM((b_per_w, D), jnp.float32),
            pltpu.VMEM((b_per_w, D), jnp.float32),
            pltpu.SemaphoreType.DMA,
            pltpu.SemaphoreType.DMA,
        ],
    )
    def body(idx_hbm, tok_hbm, pos_hbm, out_hbm,
             idx_v, grp_v, groups_v, out_v, pos_v, sem_g, sem_p):
        wid = lax.axis_index("s") * _NC + lax.axis_index("c")
        base = wid * b_per_w
        t0 = lax.rem(base, T)

        pltpu.sync_copy(idx_hbm.at[pl.ds(base, b_per_w)], idx_v)
        pos_cp = pltpu.async_copy(pos_hbm.at[pl.ds(t0, b_per_w)], pos_v, sem_p)

        for c in range(b_per_w // _L):
            s = pl.ds(c * _L, _L)
            grp_v[s] = lax.shift_right_logical(idx_v[s], 3)

        pos_cp.wait()
        for sgi in range(n_seg):
            pltpu.async_copy(
                tok_hbm.at[grp_v.at[pl.ds(sgi * _SEG, _SEG)]],
                groups_v, sem_g,
            ).wait()

            def seg_body(ci, carry, sgi=sgi):
                row0 = sgi * _SEG + ci * _L
                sub = idx_v[pl.ds(row0, _L)] & 7
                for l in range(_L):
                    r = sub[l]
                    i = row0 + l
                    for j in range(D // _L):
                        s = pl.ds(j * _L, _L)
                        out_v[i, s] = groups_v[ci * _L + l, r, s] + pos_v[i, s]
                return carry
            lax.fori_loop(0, _SEG // _L, seg_body, 0)

        pltpu.sync_copy(out_v, out_hbm.at[pl.ds(base, b_per_w)])

    return body(idx_flat, tok_grouped, pos_table)


def kernel(idx, tok_table, pos_table):
    B, T = idx.shape
    V, D = tok_table.shape
    idx_flat = idx.reshape(-1).astype(jnp.int32)
    tok_grouped = tok_table.reshape(V // 8, 8, D)
    out = _embed_lookup(idx_flat, tok_grouped, pos_table, B, T, D)
    return out.reshape(B, T, D)


# speedup vs baseline: 2.3566x; 2.3566x over previous
import functools

import jax
import jax.numpy as jnp
from jax import lax
from jax.experimental import pallas as pl
from jax.experimental.pallas import tpu as pltpu
from jax.experimental.pallas import tpu_sc as plsc

_NC = 2
_NS = 16
_NW = _NC * _NS
_L = 16
_SEG = 32


@functools.partial(jax.jit, static_argnums=(3, 4, 5))
def _embed_lookup(idx_flat, tok_grouped, pos_table, B, T, D):
    n_tok = B * T
    b_per_w = n_tok // _NW
    n_seg = b_per_w // _SEG
    mesh = plsc.VectorSubcoreMesh(core_axis_name="c", subcore_axis_name="s")

    @functools.partial(
        pl.kernel,
        out_type=jax.ShapeDtypeStruct((n_tok, D), jnp.float32),
        mesh=mesh,
        scratch_types=[
            pltpu.VMEM((b_per_w,), jnp.int32),
            pltpu.VMEM((_SEG, 8, D), jnp.float32),
            pltpu.VMEM((b_per_w, D), jnp.float32),
            pltpu.VMEM((b_per_w, D), jnp.float32),
            pltpu.SemaphoreType.DMA,
            pltpu.SemaphoreType.DMA,
        ],
    )
    def body(idx_hbm, tok_hbm, pos_hbm, out_hbm,
             idx_v, groups_v, out_v, pos_v, sem_g, sem_p):
        wid = lax.axis_index("s") * _NC + lax.axis_index("c")
        base = wid * b_per_w
        t0 = lax.rem(base, T)

        pltpu.sync_copy(idx_hbm.at[pl.ds(base, b_per_w)], idx_v)
        pos_cp = pltpu.async_copy(pos_hbm.at[pl.ds(t0, b_per_w)], pos_v, sem_p)
        pos_cp.wait()

        for sgi in range(n_seg):
            copies = []
            for ci in range(_SEG // _L):
                v = lax.shift_right_logical(
                    idx_v[pl.ds(sgi * _SEG + ci * _L, _L)], 3)
                for l in range(_L):
                    copies.append(pltpu.async_copy(
                        tok_hbm.at[pl.ds(v[l], 1)],
                        groups_v.at[pl.ds(ci * _L + l, 1)],
                        sem_g,
                    ))
            for cp in copies:
                cp.wait()

            def seg_body(ci, carry, sgi=sgi):
                row0 = sgi * _SEG + ci * _L
                sub = idx_v[pl.ds(row0, _L)] & 7
                for l in range(_L):
                    r = sub[l]
                    i = row0 + l
                    for j in range(D // _L):
                        s = pl.ds(j * _L, _L)
                        out_v[i, s] = groups_v[ci * _L + l, r, s] + pos_v[i, s]
                return carry
            lax.fori_loop(0, _SEG // _L, seg_body, 0)

        pltpu.sync_copy(out_v, out_hbm.at[pl.ds(base, b_per_w)])

    return body(idx_flat, tok_grouped, pos_table)


def kernel(idx, tok_table, pos_table):
    B, T = idx.shape
    V, D = tok_table.shape
    idx_flat = idx.reshape(-1).astype(jnp.int32)
    tok_grouped = tok_table.reshape(V // 8, 8, D)
    out = _embed_lookup(idx_flat, tok_grouped, pos_table, B, T, D)
    return out.reshape(B, T, D)


# speedup vs baseline: 2.3975x; 1.0174x over previous
import functools

import jax
import jax.numpy as jnp
from jax import lax
from jax.experimental import pallas as pl
from jax.experimental.pallas import tpu as pltpu
from jax.experimental.pallas import tpu_sc as plsc

_NC = 2
_NS = 16
_NW = _NC * _NS
_L = 16
_SEG = 16


@functools.partial(jax.jit, static_argnums=(3, 4, 5))
def _embed_lookup(idx_flat, tok_grouped, pos_table, B, T, D):
    n_tok = B * T
    b_per_w = n_tok // _NW
    n_seg = b_per_w // _SEG
    mesh = plsc.VectorSubcoreMesh(core_axis_name="c", subcore_axis_name="s")

    @functools.partial(
        pl.kernel,
        out_type=jax.ShapeDtypeStruct((n_tok, D), jnp.float32),
        mesh=mesh,
        scratch_types=[
            pltpu.VMEM((b_per_w,), jnp.int32),
            pltpu.VMEM((_SEG, 8, D), jnp.float32),
            pltpu.VMEM((_SEG, 8, D), jnp.float32),
            pltpu.VMEM((b_per_w, D), jnp.float32),
            pltpu.VMEM((b_per_w, D), jnp.float32),
            pltpu.SemaphoreType.DMA,
            pltpu.SemaphoreType.DMA,
        ],
    )
    def body(idx_hbm, tok_hbm, pos_hbm, out_hbm,
             idx_v, groups_a, groups_b, out_v, pos_v, sem_g, sem_p):
        wid = lax.axis_index("s") * _NC + lax.axis_index("c")
        base = wid * b_per_w
        t0 = lax.rem(base, T)

        pltpu.sync_copy(idx_hbm.at[pl.ds(base, b_per_w)], idx_v)
        pos_cp = pltpu.async_copy(pos_hbm.at[pl.ds(t0, b_per_w)], pos_v, sem_p)

        bufs = [groups_a, groups_b]

        def fire(sgi):
            buf = bufs[sgi % 2]
            copies = []
            for ci in range(_SEG // _L):
                v = lax.shift_right_logical(
                    idx_v[pl.ds(sgi * _SEG + ci * _L, _L)], 3)
                for l in range(_L):
                    copies.append(pltpu.async_copy(
                        tok_hbm.at[pl.ds(v[l], 1)],
                        buf.at[pl.ds(ci * _L + l, 1)],
                        sem_g,
                    ))
            return copies

        def extract(sgi):
            buf = bufs[sgi % 2]

            def seg_body(ci, carry):
                row0 = sgi * _SEG + ci * _L
                sub = idx_v[pl.ds(row0, _L)] & 7
                for l in range(_L):
                    r = sub[l]
                    i = row0 + l
                    for j in range(D // _L):
                        s = pl.ds(j * _L, _L)
                        out_v[i, s] = buf[ci * _L + l, r, s] + pos_v[i, s]
                return carry
            lax.fori_loop(0, _SEG // _L, seg_body, 0)

        pending = fire(0)
        pos_cp.wait()
        for sgi in range(n_seg):
            nxt = fire(sgi + 1) if sgi + 1 < n_seg else []
            for cp in pending:
                cp.wait()
            extract(sgi)
            pending = nxt

        pltpu.sync_copy(out_v, out_hbm.at[pl.ds(base, b_per_w)])

    return body(idx_flat, tok_grouped, pos_table)


def kernel(idx, tok_table, pos_table):
    B, T = idx.shape
    V, D = tok_table.shape
    idx_flat = idx.reshape(-1).astype(jnp.int32)
    tok_grouped = tok_table.reshape(V // 8, 8, D)
    out = _embed_lookup(idx_flat, tok_grouped, pos_table, B, T, D)
    return out.reshape(B, T, D)
